# Initial kernel scaffold; baseline (speedup 1.0000x reference)
import jax
import jax.numpy as jnp
from jax.experimental import pallas as pl


def kernel(x, edge_index, edge_attr, batch, W_emb, b_emb, We, be, Wm1, bm1, Wm2, bm2, gamma, beta, Wo1, bo1, Wo2, bo2, Wo3, bo3, Wo4, bo4):
    raise NotImplementedError("write your pallas kernel here")


# speedup vs baseline: 2.1192x; 2.1192x over previous
import functools

import jax
import jax.numpy as jnp
from jax import lax
from jax.experimental import pallas as pl
from jax.experimental.pallas import tpu as pltpu
from jax.experimental.pallas import tpu_sc as plsc

N = 10000
E = 320000
D = 128
H = 256
G = 64

NC = 2
NS = 16
NW = NC * NS
KCH = 128
NCHUNK = 80
EPT = NCHUNK * KCH
E_PAD = NW * EPT
N_ACC = 10240
RPT = N_ACC // NS
BN = 400
NB = N // BN

_f32 = jnp.float32


def _round_bf16(x):
    u = lax.bitcast_convert_type(x, jnp.uint32)
    u = u + jnp.uint32(0x7FFF) + ((u >> 16) & jnp.uint32(1))
    return lax.bitcast_convert_type(u & jnp.uint32(0xFFFF0000), _f32)


def _sc_mesh():
    return plsc.VectorSubcoreMesh(core_axis_name="c", subcore_axis_name="s")


@functools.partial(
    pl.kernel,
    out_type=jax.ShapeDtypeStruct((NC, N_ACC, D), _f32),
    mesh=_sc_mesh(),
    scratch_types=[
        pltpu.VMEM((NCHUNK, KCH), jnp.int32),
        pltpu.VMEM((NCHUNK, KCH), jnp.int32),
        pltpu.VMEM((KCH, D), _f32),
        pltpu.VMEM_SHARED((N_ACC, D), _f32),
        pltpu.SemaphoreType.DMA,
    ],
)
def _sc_seg128(h_hbm, src_hbm, dst_hbm, z_hbm, out_hbm,
               src_v, dst_v, rows_v, acc_sh, sem):
    c = lax.axis_index("c")
    s = lax.axis_index("s")
    w = c * NS + s
    r0 = s * RPT
    pltpu.sync_copy(z_hbm.at[pl.ds(r0, RPT)], acc_sh.at[pl.ds(r0, RPT)])
    pltpu.sync_copy(src_hbm.at[w], src_v)
    pltpu.sync_copy(dst_hbm.at[w], dst_v)
    plsc.subcore_barrier()

    def body(j, _):
        pltpu.async_copy(h_hbm.at[src_v.at[j]], rows_v, sem).wait()
        pltpu.sync_copy(rows_v, acc_sh.at[dst_v.at[j]], add=True)
        return 0

    lax.fori_loop(0, NCHUNK, body, 0)
    plsc.subcore_barrier()
    pltpu.sync_copy(acc_sh.at[pl.ds(r0, RPT)], out_hbm.at[c, pl.ds(r0, RPT)])


def _embed_body(x_ref, w_ref, b_ref, o_ref):
    o_ref[...] = jnp.maximum(
        jnp.dot(x_ref[...], w_ref[...], preferred_element_type=_f32)
        + b_ref[...], 0.0)


def _embed(x, w, b):
    return pl.pallas_call(
        _embed_body,
        grid=(NB,),
        in_specs=[
            pl.BlockSpec((BN, D), lambda i: (i, 0)),
            pl.BlockSpec((D, D), lambda i: (0, 0)),
            pl.BlockSpec((1, D), lambda i: (0, 0)),
        ],
        out_specs=pl.BlockSpec((BN, D), lambda i: (i, 0)),
        out_shape=jax.ShapeDtypeStruct((N, D), _f32),
    )(x, w, b)


def _layer1_body(h_ref, a0_ref, a1_ref, e0_ref, e1_ref, waug_ref,
                 w1_ref, b1_ref, w2_ref, b2_ref,
                 z_ref, sum_ref, sq_ref):
    i = pl.program_id(0)
    e0 = jnp.reshape(e0_ref[...], (BN, D))
    e1 = jnp.reshape(e1_ref[...], (BN, D))
    et = jnp.dot(e0 + e1, waug_ref[...], preferred_element_type=_f32,
                 precision=lax.Precision.HIGHEST)
    z = (h_ref[...] + jnp.reshape(a0_ref[...], (BN, D))
         + jnp.reshape(a1_ref[...], (BN, D)) + et)
    z1 = jnp.maximum(jnp.dot(z, w1_ref[...], preferred_element_type=_f32)
                     + b1_ref[...], 0.0)
    zp = jnp.dot(z1, w2_ref[...], preferred_element_type=_f32) + b2_ref[...]
    z_ref[...] = zp
    ps = jnp.sum(zp, axis=0, keepdims=True)
    pq = jnp.sum(zp * zp, axis=0, keepdims=True)

    @pl.when(i == 0)
    def _():
        sum_ref[...] = ps
        sq_ref[...] = pq

    @pl.when(i > 0)
    def _():
        sum_ref[...] += ps
        sq_ref[...] += pq


def _layer1(h, acc, eacc, waug, w1, b1, w2, b2):
    return pl.pallas_call(
        _layer1_body,
        grid=(NB,),
        in_specs=[
            pl.BlockSpec((BN, D), lambda i: (i, 0)),
            pl.BlockSpec((1, BN, D), lambda i: (0, i, 0)),
            pl.BlockSpec((1, BN, D), lambda i: (1, i, 0)),
            pl.BlockSpec((1, BN, D), lambda i: (0, i, 0)),
            pl.BlockSpec((1, BN, D), lambda i: (1, i, 0)),
            pl.BlockSpec((D, D), lambda i: (0, 0)),
            pl.BlockSpec((D, H), lambda i: (0, 0)),
            pl.BlockSpec((1, H), lambda i: (0, 0)),
            pl.BlockSpec((H, D), lambda i: (0, 0)),
            pl.BlockSpec((1, D), lambda i: (0, 0)),
        ],
        out_specs=[
            pl.BlockSpec((BN, D), lambda i: (i, 0)),
            pl.BlockSpec((1, D), lambda i: (0, 0)),
            pl.BlockSpec((1, D), lambda i: (0, 0)),
        ],
        out_shape=[
            jax.ShapeDtypeStruct((N, D), _f32),
            jax.ShapeDtypeStruct((1, D), _f32),
            jax.ShapeDtypeStruct((1, D), _f32),
        ],
    )(h, acc, acc, eacc, eacc, waug, w1, b1, w2, b2)


def _layer2_body(z_ref, sum_ref, sq_ref, g_ref, b_ref, o_ref, *, do_relu):
    mu = sum_ref[...] / N
    var = sq_ref[...] / N - mu * mu
    o = (z_ref[...] - mu) * lax.rsqrt(var + 1e-5) * g_ref[...] + b_ref[...]
    if do_relu:
        o = jnp.maximum(o, 0.0)
    o_ref[...] = o


def _layer2(z, s, q, gamma, beta, do_relu):
    return pl.pallas_call(
        functools.partial(_layer2_body, do_relu=do_relu),
        grid=(NB,),
        in_specs=[
            pl.BlockSpec((BN, D), lambda i: (i, 0)),
            pl.BlockSpec((1, D), lambda i: (0, 0)),
            pl.BlockSpec((1, D), lambda i: (0, 0)),
            pl.BlockSpec((1, D), lambda i: (0, 0)),
            pl.BlockSpec((1, D), lambda i: (0, 0)),
        ],
        out_specs=pl.BlockSpec((BN, D), lambda i: (i, 0)),
        out_shape=jax.ShapeDtypeStruct((N, D), _f32),
    )(z, s, q, gamma, beta)


def _pool_body(h_ref, b_ref, w1_ref, c1_ref, w2_ref, c2_ref,
               w3_ref, c3_ref, w4_ref, c4_ref, o_ref, s_acc, c_acc):
    i = pl.program_id(0)
    bids = jnp.reshape(b_ref[...], (1, BN))
    gi = lax.broadcasted_iota(jnp.int32, (G, BN), 0)
    oh = (gi == bids).astype(_f32)
    ps = jnp.dot(oh, h_ref[...], preferred_element_type=_f32,
                 precision=lax.Precision.HIGHEST)
    pc = jnp.dot(oh, jnp.ones((BN, D), _f32), preferred_element_type=_f32,
                 precision=lax.Precision.HIGHEST)

    @pl.when(i == 0)
    def _():
        s_acc[...] = ps
        c_acc[...] = pc

    @pl.when(i > 0)
    def _():
        s_acc[...] += ps
        c_acc[...] += pc

    @pl.when(i == NB - 1)
    def _():
        g = s_acc[...] / jnp.maximum(c_acc[...], 1.0)
        o = jnp.maximum(jnp.dot(g, w1_ref[...], preferred_element_type=_f32)
                        + c1_ref[...], 0.0)
        o = jnp.maximum(jnp.dot(o, w2_ref[...], preferred_element_type=_f32)
                        + c2_ref[...], 0.0)
        o = jnp.maximum(jnp.dot(o, w3_ref[...], preferred_element_type=_f32)
                        + c3_ref[...], 0.0)
        o_ref[...] = (jnp.dot(o, w4_ref[...], preferred_element_type=_f32)
                      + c4_ref[...])


def _pool_head(h, batch3, w1, c1, w2, c2, w3, c3, w4, c4):
    wspec = pl.BlockSpec((D, D), lambda i: (0, 0))
    bspec = pl.BlockSpec((1, D), lambda i: (0, 0))
    return pl.pallas_call(
        _pool_body,
        grid=(NB,),
        in_specs=[
            pl.BlockSpec((BN, D), lambda i: (i, 0)),
            pl.BlockSpec((1, 1, BN), lambda i: (i, 0, 0)),
            wspec, bspec, wspec, bspec, wspec, bspec, wspec, bspec,
        ],
        out_specs=pl.BlockSpec((G, D), lambda i: (0, 0)),
        out_shape=jax.ShapeDtypeStruct((G, D), _f32),
        scratch_shapes=[pltpu.VMEM((G, D), _f32), pltpu.VMEM((G, D), _f32)],
    )(h, batch3, w1, c1, w2, c2, w3, c3, w4, c4)


def kernel(x, edge_index, edge_attr, batch, W_emb, b_emb, We, be,
           Wm1, bm1, Wm2, bm2, gamma, beta,
           Wo1, bo1, Wo2, bo2, Wo3, bo3, Wo4, bo4):
    npad = E_PAD - E
    src_p = jnp.concatenate(
        [edge_index[0], jnp.zeros((npad,), jnp.int32)]).reshape(NW, NCHUNK, KCH)
    dst_p = jnp.concatenate(
        [edge_index[1], jnp.full((npad,), N, jnp.int32)]).reshape(NW, NCHUNK, KCH)
    ea_r = _round_bf16(edge_attr)
    aug = jnp.concatenate(
        [ea_r, jnp.ones((E, 1), _f32), jnp.zeros((E, D - 17), _f32)],
        axis=1)
    aug = jnp.concatenate([aug, jnp.zeros((npad, D), _f32)], axis=0)
    iota_p = jnp.arange(E_PAD, dtype=jnp.int32).reshape(NW, NCHUNK, KCH)
    z128 = jnp.zeros((N_ACC, D), _f32)

    h = _embed(x, W_emb, jnp.reshape(b_emb, (1, D)))
    eacc = _sc_seg128(aug, iota_p, dst_p, z128)

    for l in range(3):
        waug = jnp.concatenate(
            [_round_bf16(We[l]), be[l][None, :],
             jnp.zeros((D - 17, D), _f32)], axis=0)
        acc = _sc_seg128(h, src_p, dst_p, z128)
        z, s, q = _layer1(h, acc, eacc, waug, Wm1[l],
                          jnp.reshape(bm1[l], (1, H)), Wm2[l],
                          jnp.reshape(bm2[l], (1, D)))
        h = _layer2(z, s, q, jnp.reshape(gamma[l], (1, D)),
                    jnp.reshape(beta[l], (1, D)), do_relu=(l < 2))

    w2p = jnp.zeros((D, D), _f32).at[:, :64].set(Wo2)
    c2p = jnp.zeros((1, D), _f32).at[0, :64].set(bo2)
    w3p = jnp.zeros((D, D), _f32).at[:64, :32].set(Wo3)
    c3p = jnp.zeros((1, D), _f32).at[0, :32].set(bo3)
    w4p = jnp.zeros((D, D), _f32).at[:32, :1].set(Wo4)
    c4p = jnp.zeros((1, D), _f32).at[0, :1].set(bo4)
    out = _pool_head(h, batch.reshape(NB, 1, BN), Wo1,
                     jnp.reshape(bo1, (1, D)), w2p, c2p, w3p, c3p, w4p, c4p)
    return out[:, :1]


# speedup vs baseline: 2.3987x; 1.1319x over previous
import functools

import jax
import jax.numpy as jnp
from jax import lax
from jax.experimental import pallas as pl
from jax.experimental.pallas import tpu as pltpu
from jax.experimental.pallas import tpu_sc as plsc

N = 10000
E = 320000
D = 128
H = 256
G = 64

NC = 2
NS = 16
NW = NC * NS
KCH = 128
NCHUNK = 80
EPT = NCHUNK * KCH
E_PAD = NW * EPT
N_ACC = 10112
RPT = N_ACC // NS
BN = 400
NB = N // BN
BE = 4096
NBE = E_PAD // BE

_f32 = jnp.float32


@functools.partial(
    pl.kernel,
    out_type=jax.ShapeDtypeStruct((NC, N_ACC, D), _f32),
    mesh=plsc.VectorSubcoreMesh(core_axis_name="c", subcore_axis_name="s"),
    scratch_types=[
        pltpu.VMEM((2, KCH), jnp.int32),
        pltpu.VMEM((2, KCH), jnp.int32),
        pltpu.VMEM((KCH, D), _f32),
        pltpu.VMEM((KCH, D), _f32),
        pltpu.VMEM((KCH, D), _f32),
        pltpu.VMEM_SHARED((N_ACC, D), _f32),
        pltpu.SemaphoreType.DMA,
        pltpu.SemaphoreType.DMA,
        pltpu.SemaphoreType.DMA,
        pltpu.SemaphoreType.DMA,
    ],
)
def _sc_msgseg(h_hbm, em_hbm, src_hbm, dst_hbm, z_hbm, out_hbm,
               sring, dring, hb0, hb1, eb, acc_sh, semh0, semh1, semi, seme):
    c = lax.axis_index("c")
    s = lax.axis_index("s")
    w = c * NS + s
    r0 = s * RPT
    pltpu.sync_copy(z_hbm.at[pl.ds(r0, RPT)], acc_sh.at[pl.ds(r0, RPT)])
    pltpu.sync_copy(src_hbm.at[w, 0], sring.at[0])
    pltpu.sync_copy(dst_hbm.at[w, 0], dring.at[0])
    pltpu.sync_copy(src_hbm.at[w, 1], sring.at[1])
    pltpu.sync_copy(dst_hbm.at[w, 1], dring.at[1])
    plsc.subcore_barrier()

    e0 = w * EPT
    pltpu.async_copy(h_hbm.at[sring.at[0]], hb0, semh0)
    pltpu.async_copy(em_hbm.at[pl.ds(e0, KCH)], eb, seme)

    hbufs = (hb0, hb1)
    semhs = (semh0, semh1)

    def step(j, p):
        buf, nbuf = hbufs[p], hbufs[1 - p]
        sem, nsem = semhs[p], semhs[1 - p]
        pltpu.make_async_copy(em_hbm.at[pl.ds(e0 + j * KCH, KCH)], eb,
                              seme).wait()
        pltpu.make_async_copy(h_hbm.at[sring.at[p]], buf, sem).wait()

        @pl.when(j + 1 < NCHUNK)
        def _():
            @pl.when(j >= 1)
            def _():
                pltpu.make_async_copy(src_hbm.at[w, 0], sring.at[1 - p],
                                      semi).wait()
                pltpu.make_async_copy(dst_hbm.at[w, 0], dring.at[1 - p],
                                      semi).wait()
            pltpu.async_copy(h_hbm.at[sring.at[1 - p]], nbuf, nsem)

        def addrow(r, _):
            for cg in range(D // 16):
                sl = pl.ds(cg * 16, 16)
                eb[r, sl] = buf[r, sl] + eb[r, sl]
            return 0

        lax.fori_loop(0, KCH, addrow, 0)
        pltpu.sync_copy(eb, acc_sh.at[dring.at[p]], add=True)

        @pl.when(j + 2 < NCHUNK)
        def _():
            pltpu.async_copy(src_hbm.at[w, j + 2], sring.at[p], semi)
            pltpu.async_copy(dst_hbm.at[w, j + 2], dring.at[p], semi)

        @pl.when(j + 1 < NCHUNK)
        def _():
            pltpu.async_copy(em_hbm.at[pl.ds(e0 + (j + 1) * KCH, KCH)], eb,
                             seme)

    def body(jj, _):
        step(jj * 2, 0)
        step(jj * 2 + 1, 1)
        return 0

    lax.fori_loop(0, NCHUNK // 2, body, 0)
    plsc.subcore_barrier()
    pltpu.sync_copy(acc_sh.at[pl.ds(r0, RPT)], out_hbm.at[c, pl.ds(r0, RPT)])


def _emat_body(ea_ref, w0_ref, b0_ref, w1_ref, b1_ref, w2_ref, b2_ref,
               o0_ref, o1_ref, o2_ref):
    ea = ea_ref[...]
    o0_ref[...] = jnp.dot(ea, w0_ref[...], preferred_element_type=_f32) + b0_ref[...]
    o1_ref[...] = jnp.dot(ea, w1_ref[...], preferred_element_type=_f32) + b1_ref[...]
    o2_ref[...] = jnp.dot(ea, w2_ref[...], preferred_element_type=_f32) + b2_ref[...]


def _emat(ea_s, We, be):
    wspec = pl.BlockSpec((16, D), lambda i: (0, 0))
    bspec = pl.BlockSpec((1, D), lambda i: (0, 0))
    ospec = pl.BlockSpec((BE, D), lambda i: (i, 0))
    oshape = jax.ShapeDtypeStruct((E_PAD, D), _f32)
    return pl.pallas_call(
        _emat_body,
        grid=(NBE,),
        in_specs=[pl.BlockSpec((BE, 16), lambda i: (i, 0)),
                  wspec, bspec, wspec, bspec, wspec, bspec],
        out_specs=[ospec, ospec, ospec],
        out_shape=[oshape, oshape, oshape],
    )(ea_s, We[0], be[0].reshape(1, D), We[1], be[1].reshape(1, D),
      We[2], be[2].reshape(1, D))


def _embed_body(x_ref, w_ref, b_ref, o_ref):
    o_ref[...] = jnp.maximum(
        jnp.dot(x_ref[...], w_ref[...], preferred_element_type=_f32)
        + b_ref[...], 0.0)


def _embed(x, w, b):
    return pl.pallas_call(
        _embed_body,
        grid=(NB,),
        in_specs=[
            pl.BlockSpec((BN, D), lambda i: (i, 0)),
            pl.BlockSpec((D, D), lambda i: (0, 0)),
            pl.BlockSpec((1, D), lambda i: (0, 0)),
        ],
        out_specs=pl.BlockSpec((BN, D), lambda i: (i, 0)),
        out_shape=jax.ShapeDtypeStruct((N, D), _f32),
    )(x, w, b)


def _layer1_body(h_ref, a0_ref, a1_ref, w1_ref, b1_ref, w2_ref, b2_ref,
                 z_ref, sum_ref, sq_ref):
    i = pl.program_id(0)
    z = (h_ref[...] + jnp.reshape(a0_ref[...], (BN, D))
         + jnp.reshape(a1_ref[...], (BN, D)))
    z1 = jnp.maximum(jnp.dot(z, w1_ref[...], preferred_element_type=_f32)
                     + b1_ref[...], 0.0)
    zp = jnp.dot(z1, w2_ref[...], preferred_element_type=_f32) + b2_ref[...]
    z_ref[...] = zp
    ps = jnp.sum(zp, axis=0, keepdims=True)
    pq = jnp.sum(zp * zp, axis=0, keepdims=True)

    @pl.when(i == 0)
    def _():
        sum_ref[...] = ps
        sq_ref[...] = pq

    @pl.when(i > 0)
    def _():
        sum_ref[...] += ps
        sq_ref[...] += pq


def _layer1(h, acc, w1, b1, w2, b2):
    return pl.pallas_call(
        _layer1_body,
        grid=(NB,),
        in_specs=[
            pl.BlockSpec((BN, D), lambda i: (i, 0)),
            pl.BlockSpec((1, BN, D), lambda i: (0, i, 0)),
            pl.BlockSpec((1, BN, D), lambda i: (1, i, 0)),
            pl.BlockSpec((D, H), lambda i: (0, 0)),
            pl.BlockSpec((1, H), lambda i: (0, 0)),
            pl.BlockSpec((H, D), lambda i: (0, 0)),
            pl.BlockSpec((1, D), lambda i: (0, 0)),
        ],
        out_specs=[
            pl.BlockSpec((BN, D), lambda i: (i, 0)),
            pl.BlockSpec((1, D), lambda i: (0, 0)),
            pl.BlockSpec((1, D), lambda i: (0, 0)),
        ],
        out_shape=[
            jax.ShapeDtypeStruct((N, D), _f32),
            jax.ShapeDtypeStruct((1, D), _f32),
            jax.ShapeDtypeStruct((1, D), _f32),
        ],
    )(h, acc, acc, w1, b1, w2, b2)


def _layer2_body(z_ref, sum_ref, sq_ref, g_ref, b_ref, o_ref, *, do_relu):
    mu = sum_ref[...] / N
    var = sq_ref[...] / N - mu * mu
    o = (z_ref[...] - mu) * lax.rsqrt(var + 1e-5) * g_ref[...] + b_ref[...]
    if do_relu:
        o = jnp.maximum(o, 0.0)
    o_ref[...] = o


def _layer2(z, s, q, gamma, beta, do_relu):
    return pl.pallas_call(
        functools.partial(_layer2_body, do_relu=do_relu),
        grid=(NB,),
        in_specs=[
            pl.BlockSpec((BN, D), lambda i: (i, 0)),
            pl.BlockSpec((1, D), lambda i: (0, 0)),
            pl.BlockSpec((1, D), lambda i: (0, 0)),
            pl.BlockSpec((1, D), lambda i: (0, 0)),
            pl.BlockSpec((1, D), lambda i: (0, 0)),
        ],
        out_specs=pl.BlockSpec((BN, D), lambda i: (i, 0)),
        out_shape=jax.ShapeDtypeStruct((N, D), _f32),
    )(z, s, q, gamma, beta)


def _pool_body(h_ref, b_ref, w1_ref, c1_ref, w2_ref, c2_ref,
               w3_ref, c3_ref, w4_ref, c4_ref, o_ref, s_acc, c_acc):
    i = pl.program_id(0)
    bids = jnp.reshape(b_ref[...], (1, BN))
    gi = lax.broadcasted_iota(jnp.int32, (G, BN), 0)
    oh = (gi == bids).astype(_f32)
    ps = jnp.dot(oh, h_ref[...], preferred_element_type=_f32,
                 precision=lax.Precision.HIGHEST)
    pc = jnp.dot(oh, jnp.ones((BN, D), _f32), preferred_element_type=_f32,
                 precision=lax.Precision.HIGHEST)

    @pl.when(i == 0)
    def _():
        s_acc[...] = ps
        c_acc[...] = pc

    @pl.when(i > 0)
    def _():
        s_acc[...] += ps
        c_acc[...] += pc

    @pl.when(i == NB - 1)
    def _():
        g = s_acc[...] / jnp.maximum(c_acc[...], 1.0)
        o = jnp.maximum(jnp.dot(g, w1_ref[...], preferred_element_type=_f32)
                        + c1_ref[...], 0.0)
        o = jnp.maximum(jnp.dot(o, w2_ref[...], preferred_element_type=_f32)
                        + c2_ref[...], 0.0)
        o = jnp.maximum(jnp.dot(o, w3_ref[...], preferred_element_type=_f32)
                        + c3_ref[...], 0.0)
        o_ref[...] = (jnp.dot(o, w4_ref[...], preferred_element_type=_f32)
                      + c4_ref[...])


def _pool_head(h, batch3, w1, c1, w2, c2, w3, c3, w4, c4):
    wspec = pl.BlockSpec((D, D), lambda i: (0, 0))
    bspec = pl.BlockSpec((1, D), lambda i: (0, 0))
    return pl.pallas_call(
        _pool_body,
        grid=(NB,),
        in_specs=[
            pl.BlockSpec((BN, D), lambda i: (i, 0)),
            pl.BlockSpec((1, 1, BN), lambda i: (i, 0, 0)),
            wspec, bspec, wspec, bspec, wspec, bspec, wspec, bspec,
        ],
        out_specs=pl.BlockSpec((G, D), lambda i: (0, 0)),
        out_shape=jax.ShapeDtypeStruct((G, D), _f32),
        scratch_shapes=[pltpu.VMEM((G, D), _f32), pltpu.VMEM((G, D), _f32)],
    )(h, batch3, w1, c1, w2, c2, w3, c3, w4, c4)


def kernel(x, edge_index, edge_attr, batch, W_emb, b_emb, We, be,
           Wm1, bm1, Wm2, bm2, gamma, beta,
           Wo1, bo1, Wo2, bo2, Wo3, bo3, Wo4, bo4):
    npad = E_PAD - E
    src_s = edge_index[0]
    dst_s = edge_index[1]
    ea_s = edge_attr
    src_p = jnp.concatenate(
        [src_s, jnp.zeros((npad,), jnp.int32)]).reshape(NW, NCHUNK, KCH)
    dst_p = jnp.concatenate(
        [dst_s, jnp.full((npad,), N, jnp.int32)]).reshape(NW, NCHUNK, KCH)
    ea_p = jnp.concatenate([ea_s, jnp.zeros((npad, 16), _f32)], axis=0)
    z128 = jnp.zeros((N_ACC, D), _f32)

    h = _embed(x, W_emb, jnp.reshape(b_emb, (1, D)))
    em = _emat(ea_p, We, be)

    for l in range(3):
        acc = _sc_msgseg(h, em[l], src_p, dst_p, z128)
        z, s, q = _layer1(h, acc, Wm1[l], jnp.reshape(bm1[l], (1, H)),
                          Wm2[l], jnp.reshape(bm2[l], (1, D)))
        h = _layer2(z, s, q, jnp.reshape(gamma[l], (1, D)),
                    jnp.reshape(beta[l], (1, D)), do_relu=(l < 2))

    w2p = jnp.zeros((D, D), _f32).at[:, :64].set(Wo2)
    c2p = jnp.zeros((1, D), _f32).at[0, :64].set(bo2)
    w3p = jnp.zeros((D, D), _f32).at[:64, :32].set(Wo3)
    c3p = jnp.zeros((1, D), _f32).at[0, :32].set(bo3)
    w4p = jnp.zeros((D, D), _f32).at[:32, :1].set(Wo4)
    c4p = jnp.zeros((1, D), _f32).at[0, :1].set(bo4)
    out = _pool_head(h, batch.reshape(NB, 1, BN), Wo1,
                     jnp.reshape(bo1, (1, D)), w2p, c2p, w3p, c3p, w4p, c4p)
    return out[:, :1]
